# baseline (speedup 1.0000x reference)
import math

import jax
import jax.numpy as jnp
from jax.experimental import pallas as pl
from jax.experimental.pallas import tpu as pltpu

_MOMENTUM = 0.99
_LN_M = math.log(_MOMENTUM)


def _conv_feat_kernel(x_ref, wc1_ref, wc2_ref, bn_ref, out_ref, xc,
                      *, H, W, C, G):
    P = H * W
    bf16 = jnp.bfloat16

    lane = jax.lax.broadcasted_iota(jnp.int32, (C, P), 1)
    w_id = lane % W
    m_left = (w_id >= 1).astype(bf16)
    m_right = (w_id < W - 1).astype(bf16)
    taps = []
    for dh in (-1, 0, 1):
        for dw in (-1, 0, 1):
            mask = m_left if dw == -1 else (m_right if dw == 1 else None)
            taps.append((dh * W + dw, mask))

    def build(v):
        outs = []
        for (o, mask) in taps:
            if o > 0:
                sh = jnp.concatenate(
                    [v[:, o:], jnp.zeros((C, o), bf16)], axis=1)
            elif o < 0:
                sh = jnp.concatenate(
                    [jnp.zeros((C, -o), bf16), v[:, :P + o]], axis=1)
            else:
                sh = v
            if mask is not None:
                sh = sh * mask
            outs.append(sh)
        return jnp.concatenate(outs, axis=0)

    for g in range(G):
        xc[:, g * P:(g + 1) * P] = build(x_ref[g].astype(bf16))

    g1 = bn_ref[:, 0:1]
    bb1 = bn_ref[:, 1:2]
    g2 = bn_ref[:, 2:3]
    bb2 = bn_ref[:, 3:4]

    f1 = jnp.dot(wc1_ref[...], xc[...], preferred_element_type=jnp.float32)
    f1 = jax.nn.relu(f1 * g1 + bb1)

    for g in range(G):
        xc[:, g * P:(g + 1) * P] = build(f1[:, g * P:(g + 1) * P].astype(bf16))

    f2 = jnp.dot(wc2_ref[...], xc[...], preferred_element_type=jnp.float32)
    f2 = jax.nn.relu(f2 * g2 + bb2)

    for g in range(G):
        out_ref[g] = jnp.sum(f2[:, g * P:(g + 1) * P], axis=1,
                             keepdims=True) * (1.0 / P)


def _gather_kernel(rt_ref, src_ref, out_ref):
    del rt_ref
    out_ref[...] = src_ref[:, :, :, 0]


def _stream_kernel(xf_ref, w1_ref, b1_ref, w2_ref, b2_ref, rf_ref,
                   rt_row_ref, rt_col_ref, refx_ref, old_ref,
                   out_ref, loss_ref,
                   m_s, s_s, acc, tl, h_s, *, K, NC, R, B, C):
    i = pl.program_id(0)

    @pl.when(i == 0)
    def _init():
        m_s[...] = jnp.full_like(m_s, -1e30)
        s_s[...] = jnp.zeros_like(s_s)
        acc[...] = jnp.zeros_like(acc)
        tl[...] = jnp.zeros_like(tl)
        h_s[...] = jax.nn.relu(
            jnp.dot(xf_ref[...], w1_ref[...],
                    preferred_element_type=jnp.float32) + b1_ref[...])

    h = h_s[...]
    logits = jnp.dot(h, w2_ref[...],
                     preferred_element_type=jnp.float32) + b2_ref[...]

    rem = R - i * K
    lane_row = jax.lax.broadcasted_iota(jnp.int32, (1, K), 1)
    logits = jnp.where(lane_row < rem, logits, -1e30)
    lane_col = jax.lax.broadcasted_iota(jnp.int32, (K, 1), 0)
    rfb = jnp.where(lane_col < rem, rf_ref[...], 0.0)

    mc = jnp.max(logits, axis=1, keepdims=True)
    m_new = jnp.maximum(m_s[...], mc)
    alpha = jnp.exp(m_s[...] - m_new)
    e = jnp.exp(logits - m_new)
    s_new = s_s[...] * alpha + jnp.sum(e, axis=1, keepdims=True)
    acc[...] = acc[...] * alpha + jnp.dot(e, rfb,
                                          preferred_element_type=jnp.float32)
    m_s[...] = m_new
    s_s[...] = s_new

    col = rt_row_ref[...] - i * K
    kio = jax.lax.broadcasted_iota(jnp.int32, (K, B), 0)
    maskT = (kio == col).astype(jnp.float32)
    tl[...] = tl[...] + jnp.dot(logits, maskT,
                                preferred_element_type=jnp.float32)

    @pl.when(i == NC - 1)
    def _finalize():
        lse = m_new + jnp.log(s_new)
        wold = acc[...] / s_new

        rt_r = rt_row_ref[...]
        rt_c = rt_col_ref[...]
        E = (rt_c == rt_r).astype(jnp.float32)
        jr = jax.lax.broadcasted_iota(jnp.int32, (B, B), 0)
        jc = jax.lax.broadcasted_iota(jnp.int32, (B, B), 1)
        after = (jr > jc).astype(jnp.float32)
        before = (jr < jc).astype(jnp.float32)

        ca_col = jnp.sum(E * after, axis=0, keepdims=True)
        cnt = jnp.sum(E, axis=1, keepdims=True)
        first_col = (jnp.sum(E * before, axis=0, keepdims=True)
                     == 0.0).astype(jnp.float32)

        coef = (1.0 - _MOMENTUM) * jnp.exp(ca_col * _LN_M)
        sum_term = jnp.dot(E * coef, refx_ref[...],
                           preferred_element_type=jnp.float32)
        dec = jnp.exp(cnt * _LN_M)
        delta = (dec - 1.0) * old_ref[...] + sum_term

        tw_t = jnp.exp(tl[...] - lse)
        corr = jnp.dot(tw_t * first_col, delta,
                       preferred_element_type=jnp.float32)

        out_ref[...] = wold + corr + refx_ref[...]

        diag = jnp.sum(tl[...] * (jr == jc).astype(jnp.float32),
                       axis=1, keepdims=True)
        loss_ref[...] = jnp.sum(lse - diag, axis=0,
                                keepdims=True) * (1.0 / B)


def kernel(x, ref_x, ref_types, ref_proj, conv1_w, bn1_g, bn1_b,
           conv2_w, bn2_g, bn2_b, w1, b1, w2, b2):
    B, C, H, W = x.shape
    R = w2.shape[1]
    C2 = w1.shape[1]
    P = H * W
    f32 = jnp.float32
    bf16 = jnp.bfloat16

    xf_in = x.reshape(B, C, P)

    inv = 1.0 / math.sqrt(1.0 + 1e-5)
    wc1 = jnp.transpose(conv1_w, (0, 2, 3, 1)).reshape(C, 9 * C).astype(bf16)
    wc2 = jnp.transpose(conv2_w, (0, 2, 3, 1)).reshape(C, 9 * C).astype(bf16)
    bn = jnp.stack([bn1_g * inv, bn1_b, bn2_g * inv, bn2_b], axis=1)

    G = 8
    xfeat3 = pl.pallas_call(
        lambda a, b_, c_, d_, o, s1: _conv_feat_kernel(
            a, b_, c_, d_, o, s1, H=H, W=W, C=C, G=G),
        grid=(B // G,),
        in_specs=[
            pl.BlockSpec((G, C, P), lambda i: (i, 0, 0)),
            pl.BlockSpec((C, 9 * C), lambda i: (0, 0)),
            pl.BlockSpec((C, 9 * C), lambda i: (0, 0)),
            pl.BlockSpec((C, 4), lambda i: (0, 0)),
        ],
        out_specs=pl.BlockSpec((G, C, 1), lambda i: (i, 0, 0)),
        out_shape=jax.ShapeDtypeStruct((B, C, 1), f32),
        scratch_shapes=[
            pltpu.VMEM((9 * C, G * P), bf16),
        ],
    )(xf_in, wc1, wc2, bn)
    x_feat = xfeat3.reshape(B, C)

    old3 = pl.pallas_call(
        _gather_kernel,
        grid_spec=pltpu.PrefetchScalarGridSpec(
            num_scalar_prefetch=1,
            grid=(B,),
            in_specs=[
                pl.BlockSpec((1, C, 1, 1), lambda i, rt: (rt[i], 0, 0, 0))],
            out_specs=pl.BlockSpec((1, C, 1), lambda i, rt: (i, 0, 0)),
        ),
        out_shape=jax.ShapeDtypeStruct((B, C, 1), f32),
    )(ref_types, ref_proj)
    old_rows = old3.reshape(B, C)

    K = 8192
    NC = pl.cdiv(R, K)

    rt_row = ref_types.reshape(1, B).astype(jnp.int32)
    rt_col = ref_types.reshape(B, 1).astype(jnp.int32)
    refx_flat = ref_x.reshape(B, C)
    b1r = b1.reshape(1, C2)
    b2r = b2.reshape(1, R)

    out1, loss2 = pl.pallas_call(
        lambda *a: _stream_kernel(*a, K=K, NC=NC, R=R, B=B, C=C),
        grid=(NC,),
        in_specs=[
            pl.BlockSpec((B, C), lambda i: (0, 0)),
            pl.BlockSpec((C, C2), lambda i: (0, 0)),
            pl.BlockSpec((1, C2), lambda i: (0, 0)),
            pl.BlockSpec((C2, K), lambda i: (0, i)),
            pl.BlockSpec((1, K), lambda i: (0, i)),
            pl.BlockSpec((K, C), lambda i: (i, 0)),
            pl.BlockSpec((1, B), lambda i: (0, 0)),
            pl.BlockSpec((B, 1), lambda i: (0, 0)),
            pl.BlockSpec((B, C), lambda i: (0, 0)),
            pl.BlockSpec((B, C), lambda i: (0, 0)),
        ],
        out_specs=[
            pl.BlockSpec((B, C), lambda i: (0, 0)),
            pl.BlockSpec((1, 1), lambda i: (0, 0)),
        ],
        out_shape=[
            jax.ShapeDtypeStruct((B, C), f32),
            jax.ShapeDtypeStruct((1, 1), f32),
        ],
        scratch_shapes=[
            pltpu.VMEM((B, 1), f32),
            pltpu.VMEM((B, 1), f32),
            pltpu.VMEM((B, C), f32),
            pltpu.VMEM((B, B), f32),
            pltpu.VMEM((B, C2), f32),
        ],
    )(x_feat, w1, b1r, w2, b2r, ref_proj.reshape(R, C), rt_row, rt_col,
      refx_flat, old_rows)

    return (out1.reshape(B, C, 1, 1), loss2[0, 0])


# speedup vs baseline: 23.1531x; 23.1531x over previous
import math

import jax
import jax.numpy as jnp
from jax.experimental import pallas as pl
from jax.experimental.pallas import tpu as pltpu

_MOMENTUM = 0.99
_LN_M = math.log(_MOMENTUM)


def _conv_feat_kernel(x_ref, wc1_ref, wc2_ref, bn_ref, out_ref, xc,
                      *, H, W, C, G):
    P = H * W
    bf16 = jnp.bfloat16

    lane = jax.lax.broadcasted_iota(jnp.int32, (C, P), 1)
    w_id = lane % W
    m_left = (w_id >= 1).astype(bf16)
    m_right = (w_id < W - 1).astype(bf16)
    taps = []
    for dh in (-1, 0, 1):
        for dw in (-1, 0, 1):
            mask = m_left if dw == -1 else (m_right if dw == 1 else None)
            taps.append((dh * W + dw, mask))

    def build(v):
        outs = []
        for (o, mask) in taps:
            if o > 0:
                sh = jnp.concatenate(
                    [v[:, o:], jnp.zeros((C, o), bf16)], axis=1)
            elif o < 0:
                sh = jnp.concatenate(
                    [jnp.zeros((C, -o), bf16), v[:, :P + o]], axis=1)
            else:
                sh = v
            if mask is not None:
                sh = sh * mask
            outs.append(sh)
        return jnp.concatenate(outs, axis=0)

    for g in range(G):
        xc[:, g * P:(g + 1) * P] = build(x_ref[g].astype(bf16))

    g1 = bn_ref[:, 0:1]
    bb1 = bn_ref[:, 1:2]
    g2 = bn_ref[:, 2:3]
    bb2 = bn_ref[:, 3:4]

    f1 = jnp.dot(wc1_ref[...], xc[...], preferred_element_type=jnp.float32)
    f1 = jax.nn.relu(f1 * g1 + bb1)

    for g in range(G):
        xc[:, g * P:(g + 1) * P] = build(f1[:, g * P:(g + 1) * P].astype(bf16))

    f2 = jnp.dot(wc2_ref[...], xc[...], preferred_element_type=jnp.float32)
    f2 = jax.nn.relu(f2 * g2 + bb2)

    for g in range(G):
        out_ref[g] = jnp.sum(f2[:, g * P:(g + 1) * P], axis=1,
                             keepdims=True) * (1.0 / P)


def _gather_kernel(rt_ref, src_ref, out_ref):
    i = pl.program_id(0)
    r = rt_ref[i] % 8
    out_ref[0] = src_ref[pl.ds(r, 1), :]


def _stream_kernel(xf_ref, w1_ref, b1_ref, w2_ref, b2_ref, rf_ref,
                   rt_row_ref, rt_col_ref, refx_ref, old_ref,
                   out_ref, loss_ref,
                   m_s, s_s, acc, tl, h_s, *, K, NC, R, B, C):
    i = pl.program_id(0)

    @pl.when(i == 0)
    def _init():
        m_s[...] = jnp.full_like(m_s, -1e30)
        s_s[...] = jnp.zeros_like(s_s)
        acc[...] = jnp.zeros_like(acc)
        tl[...] = jnp.zeros_like(tl)
        h_s[...] = jax.nn.relu(
            jnp.dot(xf_ref[...], w1_ref[...],
                    preferred_element_type=jnp.float32) + b1_ref[...])

    h = h_s[...]
    logits = jnp.dot(h, w2_ref[...],
                     preferred_element_type=jnp.float32) + b2_ref[...]

    rem = R - i * K
    lane_row = jax.lax.broadcasted_iota(jnp.int32, (1, K), 1)
    logits = jnp.where(lane_row < rem, logits, -1e30)
    lane_col = jax.lax.broadcasted_iota(jnp.int32, (K, 1), 0)
    rfb = jnp.where(lane_col < rem, rf_ref[...], 0.0)

    mc = jnp.max(logits, axis=1, keepdims=True)
    m_new = jnp.maximum(m_s[...], mc)
    alpha = jnp.exp(m_s[...] - m_new)
    e = jnp.exp(logits - m_new)
    s_new = s_s[...] * alpha + jnp.sum(e, axis=1, keepdims=True)
    acc[...] = acc[...] * alpha + jnp.dot(e, rfb,
                                          preferred_element_type=jnp.float32)
    m_s[...] = m_new
    s_s[...] = s_new

    col = rt_row_ref[...] - i * K
    kio = jax.lax.broadcasted_iota(jnp.int32, (K, B), 0)
    maskT = (kio == col).astype(jnp.float32)
    tl[...] = tl[...] + jnp.dot(logits, maskT,
                                preferred_element_type=jnp.float32)

    @pl.when(i == NC - 1)
    def _finalize():
        lse = m_new + jnp.log(s_new)
        wold = acc[...] / s_new

        rt_r = rt_row_ref[...]
        rt_c = rt_col_ref[...]
        E = (rt_c == rt_r).astype(jnp.float32)
        jr = jax.lax.broadcasted_iota(jnp.int32, (B, B), 0)
        jc = jax.lax.broadcasted_iota(jnp.int32, (B, B), 1)
        after = (jr > jc).astype(jnp.float32)
        before = (jr < jc).astype(jnp.float32)

        ca_col = jnp.sum(E * after, axis=0, keepdims=True)
        cnt = jnp.sum(E, axis=1, keepdims=True)
        first_col = (jnp.sum(E * before, axis=0, keepdims=True)
                     == 0.0).astype(jnp.float32)

        coef = (1.0 - _MOMENTUM) * jnp.exp(ca_col * _LN_M)
        sum_term = jnp.dot(E * coef, refx_ref[...],
                           preferred_element_type=jnp.float32)
        dec = jnp.exp(cnt * _LN_M)
        delta = (dec - 1.0) * old_ref[...] + sum_term

        tw_t = jnp.exp(tl[...] - lse)
        corr = jnp.dot(tw_t * first_col, delta,
                       preferred_element_type=jnp.float32)

        out_ref[...] = wold + corr + refx_ref[...]

        diag = jnp.sum(tl[...] * (jr == jc).astype(jnp.float32),
                       axis=1, keepdims=True)
        loss_ref[...] = jnp.sum(lse - diag, axis=0,
                                keepdims=True) * (1.0 / B)


def kernel(x, ref_x, ref_types, ref_proj, conv1_w, bn1_g, bn1_b,
           conv2_w, bn2_g, bn2_b, w1, b1, w2, b2):
    B, C, H, W = x.shape
    R = w2.shape[1]
    C2 = w1.shape[1]
    P = H * W
    f32 = jnp.float32
    bf16 = jnp.bfloat16

    xf_in = x.reshape(B, C, P)

    inv = 1.0 / math.sqrt(1.0 + 1e-5)
    wc1 = jnp.transpose(conv1_w, (0, 2, 3, 1)).reshape(C, 9 * C).astype(bf16)
    wc2 = jnp.transpose(conv2_w, (0, 2, 3, 1)).reshape(C, 9 * C).astype(bf16)
    bn = jnp.stack([bn1_g * inv, bn1_b, bn2_g * inv, bn2_b], axis=1)

    G = 8
    xfeat3 = pl.pallas_call(
        lambda a, b_, c_, d_, o, s1: _conv_feat_kernel(
            a, b_, c_, d_, o, s1, H=H, W=W, C=C, G=G),
        grid=(B // G,),
        in_specs=[
            pl.BlockSpec((G, C, P), lambda i: (i, 0, 0)),
            pl.BlockSpec((C, 9 * C), lambda i: (0, 0)),
            pl.BlockSpec((C, 9 * C), lambda i: (0, 0)),
            pl.BlockSpec((C, 4), lambda i: (0, 0)),
        ],
        out_specs=pl.BlockSpec((G, C, 1), lambda i: (i, 0, 0)),
        out_shape=jax.ShapeDtypeStruct((B, C, 1), f32),
        scratch_shapes=[
            pltpu.VMEM((9 * C, G * P), bf16),
        ],
    )(xf_in, wc1, wc2, bn)
    x_feat = xfeat3.reshape(B, C)

    rf = ref_proj.reshape(R, C)
    old3 = pl.pallas_call(
        _gather_kernel,
        grid_spec=pltpu.PrefetchScalarGridSpec(
            num_scalar_prefetch=1,
            grid=(B,),
            in_specs=[
                pl.BlockSpec((8, C), lambda i, rt: (rt[i] // 8, 0))],
            out_specs=pl.BlockSpec((1, 1, C), lambda i, rt: (i, 0, 0)),
        ),
        out_shape=jax.ShapeDtypeStruct((B, 1, C), f32),
    )(ref_types, rf)
    old_rows = old3.reshape(B, C)

    K = 8192
    NC = pl.cdiv(R, K)

    rt_row = ref_types.reshape(1, B).astype(jnp.int32)
    rt_col = ref_types.reshape(B, 1).astype(jnp.int32)
    refx_flat = ref_x.reshape(B, C)
    b1r = b1.reshape(1, C2)
    b2r = b2.reshape(1, R)

    out1, loss2 = pl.pallas_call(
        lambda *a: _stream_kernel(*a, K=K, NC=NC, R=R, B=B, C=C),
        grid=(NC,),
        in_specs=[
            pl.BlockSpec((B, C), lambda i: (0, 0)),
            pl.BlockSpec((C, C2), lambda i: (0, 0)),
            pl.BlockSpec((1, C2), lambda i: (0, 0)),
            pl.BlockSpec((C2, K), lambda i: (0, i)),
            pl.BlockSpec((1, K), lambda i: (0, i)),
            pl.BlockSpec((K, C), lambda i: (i, 0)),
            pl.BlockSpec((1, B), lambda i: (0, 0)),
            pl.BlockSpec((B, 1), lambda i: (0, 0)),
            pl.BlockSpec((B, C), lambda i: (0, 0)),
            pl.BlockSpec((B, C), lambda i: (0, 0)),
        ],
        out_specs=[
            pl.BlockSpec((B, C), lambda i: (0, 0)),
            pl.BlockSpec((1, 1), lambda i: (0, 0)),
        ],
        out_shape=[
            jax.ShapeDtypeStruct((B, C), f32),
            jax.ShapeDtypeStruct((1, 1), f32),
        ],
        scratch_shapes=[
            pltpu.VMEM((B, 1), f32),
            pltpu.VMEM((B, 1), f32),
            pltpu.VMEM((B, C), f32),
            pltpu.VMEM((B, B), f32),
            pltpu.VMEM((B, C2), f32),
        ],
    )(x_feat, w1, b1r, w2, b2r, rf, rt_row, rt_col, refx_flat, old_rows)

    return (out1.reshape(B, C, 1, 1), loss2[0, 0])


# speedup vs baseline: 24.4368x; 1.0554x over previous
import math

import jax
import jax.numpy as jnp
from jax.experimental import pallas as pl
from jax.experimental.pallas import tpu as pltpu

_MOMENTUM = 0.99
_LN_M = math.log(_MOMENTUM)


def _conv_feat_kernel(x_ref, wc1_ref, wc2_ref, bn_ref, out_ref, xc,
                      *, H, W, C, G):
    P = H * W
    bf16 = jnp.bfloat16

    lane = jax.lax.broadcasted_iota(jnp.int32, (C, P), 1)
    w_id = lane % W
    m_left = (w_id >= 1).astype(jnp.float32)
    m_right = (w_id < W - 1).astype(jnp.float32)

    def build_v(v):
        up = jnp.concatenate(
            [jnp.zeros((C, W), bf16), v[:, :P - W]], axis=1)
        dn = jnp.concatenate(
            [v[:, W:], jnp.zeros((C, W), bf16)], axis=1)
        return jnp.concatenate([up, v, dn], axis=0)

    def combine_h(z):
        zl = jnp.concatenate(
            [jnp.zeros((C, 1), jnp.float32), z[0:C, :P - 1]], axis=1)
        zr = jnp.concatenate(
            [z[2 * C:3 * C, 1:], jnp.zeros((C, 1), jnp.float32)], axis=1)
        return z[C:2 * C] + m_left * zl + m_right * zr

    for g in range(G):
        xc[:, g * P:(g + 1) * P] = build_v(x_ref[g].astype(bf16))

    g1 = bn_ref[:, 0:1]
    bb1 = bn_ref[:, 1:2]
    g2 = bn_ref[:, 2:3]
    bb2 = bn_ref[:, 3:4]

    z1 = jnp.dot(wc1_ref[...], xc[...], preferred_element_type=jnp.float32)
    for g in range(G):
        f1g = combine_h(z1[:, g * P:(g + 1) * P])
        f1g = jax.nn.relu(f1g * g1 + bb1)
        xc[:, g * P:(g + 1) * P] = build_v(f1g.astype(bf16))

    z2 = jnp.dot(wc2_ref[...], xc[...], preferred_element_type=jnp.float32)
    for g in range(G):
        f2g = combine_h(z2[:, g * P:(g + 1) * P])
        f2g = jax.nn.relu(f2g * g2 + bb2)
        out_ref[g] = jnp.sum(f2g, axis=1, keepdims=True) * (1.0 / P)


def _gather_kernel(rt_ref, src_ref, out_ref):
    i = pl.program_id(0)
    r = rt_ref[i] % 8
    out_ref[0] = src_ref[pl.ds(r, 1), :]


def _stream_kernel(xf_ref, w1_ref, b1_ref, w2_ref, b2_ref, rf_ref,
                   rt_row_ref, rt_col_ref, refx_ref, old_ref,
                   out_ref, loss_ref,
                   m_s, s_s, acc, tl, h_s, *, K, NC, R, B, C):
    i = pl.program_id(0)

    @pl.when(i == 0)
    def _init():
        m_s[...] = jnp.full_like(m_s, -1e30)
        s_s[...] = jnp.zeros_like(s_s)
        acc[...] = jnp.zeros_like(acc)
        tl[...] = jnp.zeros_like(tl)
        h_s[...] = jax.nn.relu(
            jnp.dot(xf_ref[...], w1_ref[...],
                    preferred_element_type=jnp.float32) + b1_ref[...])

    h = h_s[...]
    logits = jnp.dot(h, w2_ref[...],
                     preferred_element_type=jnp.float32) + b2_ref[...]

    rem = R - i * K
    lane_row = jax.lax.broadcasted_iota(jnp.int32, (1, K), 1)
    logits = jnp.where(lane_row < rem, logits, -1e30)
    lane_col = jax.lax.broadcasted_iota(jnp.int32, (K, 1), 0)
    rfb = jnp.where(lane_col < rem, rf_ref[...], 0.0)

    mc = jnp.max(logits, axis=1, keepdims=True)
    m_new = jnp.maximum(m_s[...], mc)
    alpha = jnp.exp(m_s[...] - m_new)
    e = jnp.exp(logits - m_new)
    s_new = s_s[...] * alpha + jnp.sum(e, axis=1, keepdims=True)
    acc[...] = acc[...] * alpha + jnp.dot(e, rfb,
                                          preferred_element_type=jnp.float32)
    m_s[...] = m_new
    s_s[...] = s_new

    col = rt_row_ref[...] - i * K
    kio = jax.lax.broadcasted_iota(jnp.int32, (K, B), 0)
    maskT = (kio == col).astype(jnp.float32)
    tl[...] = tl[...] + jnp.dot(logits, maskT,
                                preferred_element_type=jnp.float32)

    @pl.when(i == NC - 1)
    def _finalize():
        lse = m_new + jnp.log(s_new)
        wold = acc[...] / s_new

        rt_r = rt_row_ref[...]
        rt_c = rt_col_ref[...]
        E = (rt_c == rt_r).astype(jnp.float32)
        jr = jax.lax.broadcasted_iota(jnp.int32, (B, B), 0)
        jc = jax.lax.broadcasted_iota(jnp.int32, (B, B), 1)
        after = (jr > jc).astype(jnp.float32)
        before = (jr < jc).astype(jnp.float32)

        ca_col = jnp.sum(E * after, axis=0, keepdims=True)
        cnt = jnp.sum(E, axis=1, keepdims=True)
        first_col = (jnp.sum(E * before, axis=0, keepdims=True)
                     == 0.0).astype(jnp.float32)

        coef = (1.0 - _MOMENTUM) * jnp.exp(ca_col * _LN_M)
        sum_term = jnp.dot(E * coef, refx_ref[...],
                           preferred_element_type=jnp.float32)
        dec = jnp.exp(cnt * _LN_M)
        delta = (dec - 1.0) * old_ref[...] + sum_term

        tw_t = jnp.exp(tl[...] - lse)
        corr = jnp.dot(tw_t * first_col, delta,
                       preferred_element_type=jnp.float32)

        out_ref[...] = wold + corr + refx_ref[...]

        diag = jnp.sum(tl[...] * (jr == jc).astype(jnp.float32),
                       axis=1, keepdims=True)
        loss_ref[...] = jnp.sum(lse - diag, axis=0,
                                keepdims=True) * (1.0 / B)


def kernel(x, ref_x, ref_types, ref_proj, conv1_w, bn1_g, bn1_b,
           conv2_w, bn2_g, bn2_b, w1, b1, w2, b2):
    B, C, H, W = x.shape
    R = w2.shape[1]
    C2 = w1.shape[1]
    P = H * W
    f32 = jnp.float32
    bf16 = jnp.bfloat16

    xf_in = x.reshape(B, C, P)

    inv = 1.0 / math.sqrt(1.0 + 1e-5)
    wc1 = jnp.transpose(conv1_w, (3, 0, 2, 1)).reshape(3 * C, 3 * C)
    wc1 = wc1.astype(bf16)
    wc2 = jnp.transpose(conv2_w, (3, 0, 2, 1)).reshape(3 * C, 3 * C)
    wc2 = wc2.astype(bf16)
    bn = jnp.stack([bn1_g * inv, bn1_b, bn2_g * inv, bn2_b], axis=1)

    G = 8
    xfeat3 = pl.pallas_call(
        lambda a, b_, c_, d_, o, s1: _conv_feat_kernel(
            a, b_, c_, d_, o, s1, H=H, W=W, C=C, G=G),
        grid=(B // G,),
        in_specs=[
            pl.BlockSpec((G, C, P), lambda i: (i, 0, 0)),
            pl.BlockSpec((3 * C, 3 * C), lambda i: (0, 0)),
            pl.BlockSpec((3 * C, 3 * C), lambda i: (0, 0)),
            pl.BlockSpec((C, 4), lambda i: (0, 0)),
        ],
        out_specs=pl.BlockSpec((G, C, 1), lambda i: (i, 0, 0)),
        out_shape=jax.ShapeDtypeStruct((B, C, 1), f32),
        scratch_shapes=[
            pltpu.VMEM((3 * C, G * P), bf16),
        ],
    )(xf_in, wc1, wc2, bn)
    x_feat = xfeat3.reshape(B, C)

    rf = ref_proj.reshape(R, C)
    old3 = pl.pallas_call(
        _gather_kernel,
        grid_spec=pltpu.PrefetchScalarGridSpec(
            num_scalar_prefetch=1,
            grid=(B,),
            in_specs=[
                pl.BlockSpec((8, C), lambda i, rt: (rt[i] // 8, 0))],
            out_specs=pl.BlockSpec((1, 1, C), lambda i, rt: (i, 0, 0)),
        ),
        out_shape=jax.ShapeDtypeStruct((B, 1, C), f32),
    )(ref_types, rf)
    old_rows = old3.reshape(B, C)

    K = 8192
    NC = pl.cdiv(R, K)

    rt_row = ref_types.reshape(1, B).astype(jnp.int32)
    rt_col = ref_types.reshape(B, 1).astype(jnp.int32)
    refx_flat = ref_x.reshape(B, C)
    b1r = b1.reshape(1, C2)
    b2r = b2.reshape(1, R)

    out1, loss2 = pl.pallas_call(
        lambda *a: _stream_kernel(*a, K=K, NC=NC, R=R, B=B, C=C),
        grid=(NC,),
        in_specs=[
            pl.BlockSpec((B, C), lambda i: (0, 0)),
            pl.BlockSpec((C, C2), lambda i: (0, 0)),
            pl.BlockSpec((1, C2), lambda i: (0, 0)),
            pl.BlockSpec((C2, K), lambda i: (0, i)),
            pl.BlockSpec((1, K), lambda i: (0, i)),
            pl.BlockSpec((K, C), lambda i: (i, 0)),
            pl.BlockSpec((1, B), lambda i: (0, 0)),
            pl.BlockSpec((B, 1), lambda i: (0, 0)),
            pl.BlockSpec((B, C), lambda i: (0, 0)),
            pl.BlockSpec((B, C), lambda i: (0, 0)),
        ],
        out_specs=[
            pl.BlockSpec((B, C), lambda i: (0, 0)),
            pl.BlockSpec((1, 1), lambda i: (0, 0)),
        ],
        out_shape=[
            jax.ShapeDtypeStruct((B, C), f32),
            jax.ShapeDtypeStruct((1, 1), f32),
        ],
        scratch_shapes=[
            pltpu.VMEM((B, 1), f32),
            pltpu.VMEM((B, 1), f32),
            pltpu.VMEM((B, C), f32),
            pltpu.VMEM((B, B), f32),
            pltpu.VMEM((B, C2), f32),
        ],
    )(x_feat, w1, b1r, w2, b2r, rf, rt_row, rt_col, refx_flat, old_rows)

    return (out1.reshape(B, C, 1, 1), loss2[0, 0])


# speedup vs baseline: 26.8968x; 1.1007x over previous
import math

import jax
import jax.numpy as jnp
from jax.experimental import pallas as pl
from jax.experimental.pallas import tpu as pltpu

_MOMENTUM = 0.99
_LN_M = math.log(_MOMENTUM)


def _conv_feat_kernel(x_ref, wc1_ref, wc2_ref, bn_ref, out_ref, xc,
                      *, H, W, C, G):
    P = H * W
    bf16 = jnp.bfloat16

    lane = jax.lax.broadcasted_iota(jnp.int32, (C, P), 1)
    w_id = lane % W
    m_left = (w_id >= 1).astype(jnp.float32)
    m_right = (w_id < W - 1).astype(jnp.float32)

    def build_v(v):
        up = jnp.concatenate(
            [jnp.zeros((C, W), bf16), v[:, :P - W]], axis=1)
        dn = jnp.concatenate(
            [v[:, W:], jnp.zeros((C, W), bf16)], axis=1)
        return jnp.concatenate([up, v, dn], axis=0)

    def combine_h(z):
        zl = jnp.concatenate(
            [jnp.zeros((C, 1), jnp.float32), z[0:C, :P - 1]], axis=1)
        zr = jnp.concatenate(
            [z[2 * C:3 * C, 1:], jnp.zeros((C, 1), jnp.float32)], axis=1)
        return z[C:2 * C] + m_left * zl + m_right * zr

    for g in range(G):
        xc[:, g * P:(g + 1) * P] = build_v(x_ref[g].astype(bf16))

    g1 = bn_ref[:, 0:1]
    bb1 = bn_ref[:, 1:2]
    g2 = bn_ref[:, 2:3]
    bb2 = bn_ref[:, 3:4]

    z1 = jnp.dot(wc1_ref[...], xc[...], preferred_element_type=jnp.float32)
    for g in range(G):
        f1g = combine_h(z1[:, g * P:(g + 1) * P])
        f1g = jax.nn.relu(f1g * g1 + bb1)
        xc[:, g * P:(g + 1) * P] = build_v(f1g.astype(bf16))

    z2 = jnp.dot(wc2_ref[...], xc[...], preferred_element_type=jnp.float32)
    for g in range(G):
        f2g = combine_h(z2[:, g * P:(g + 1) * P])
        f2g = jax.nn.relu(f2g * g2 + bb2)
        out_ref[g] = jnp.sum(f2g, axis=1, keepdims=True) * (1.0 / P)


def _gather_kernel(rt_ref, *refs):
    i = pl.program_id(0)
    srcs = refs[:-1]
    out_ref = refs[-1]
    for j, src in enumerate(srcs):
        r = rt_ref[i * len(srcs) + j] % 8
        out_ref[0, j] = src[pl.ds(r, 1), :][0]


def _stream_kernel(xf_ref, w1_ref, b1_ref, w2_ref, b2_ref, rf_ref,
                   rt_row_ref, rt_col_ref, refx_ref, old_ref,
                   out_ref, loss_ref,
                   m_s, s_s, acc, tl, h_s, *, K, NC, R, B, C):
    i = pl.program_id(0)

    @pl.when(i == 0)
    def _init():
        m_s[...] = jnp.full_like(m_s, -1e30)
        s_s[...] = jnp.zeros_like(s_s)
        acc[...] = jnp.zeros_like(acc)
        tl[...] = jnp.zeros_like(tl)
        h_s[...] = jax.nn.relu(
            jnp.dot(xf_ref[...], w1_ref[...],
                    preferred_element_type=jnp.float32) + b1_ref[...])

    h = h_s[...]
    logits = jnp.dot(h, w2_ref[...],
                     preferred_element_type=jnp.float32) + b2_ref[...]

    rem = R - i * K
    lane_row = jax.lax.broadcasted_iota(jnp.int32, (1, K), 1)
    logits = jnp.where(lane_row < rem, logits, -1e30)
    lane_col = jax.lax.broadcasted_iota(jnp.int32, (K, 1), 0)
    rfb = jnp.where(lane_col < rem, rf_ref[...], 0.0)

    mc = jnp.max(logits, axis=1, keepdims=True)
    m_new = jnp.maximum(m_s[...], mc)
    alpha = jnp.exp(m_s[...] - m_new)
    e = jnp.exp(logits - m_new)
    s_new = s_s[...] * alpha + jnp.sum(e, axis=1, keepdims=True)
    acc[...] = acc[...] * alpha + jnp.dot(e, rfb,
                                          preferred_element_type=jnp.float32)
    m_s[...] = m_new
    s_s[...] = s_new

    col = rt_row_ref[...] - i * K
    kio = jax.lax.broadcasted_iota(jnp.int32, (K, B), 0)
    maskT = (kio == col).astype(jnp.float32)
    tl[...] = tl[...] + jnp.dot(logits, maskT,
                                preferred_element_type=jnp.float32)

    @pl.when(i == NC - 1)
    def _finalize():
        lse = m_new + jnp.log(s_new)
        wold = acc[...] / s_new

        rt_r = rt_row_ref[...]
        rt_c = rt_col_ref[...]
        E = (rt_c == rt_r).astype(jnp.float32)
        jr = jax.lax.broadcasted_iota(jnp.int32, (B, B), 0)
        jc = jax.lax.broadcasted_iota(jnp.int32, (B, B), 1)
        after = (jr > jc).astype(jnp.float32)
        before = (jr < jc).astype(jnp.float32)

        ca_col = jnp.sum(E * after, axis=0, keepdims=True)
        cnt = jnp.sum(E, axis=1, keepdims=True)
        first_col = (jnp.sum(E * before, axis=0, keepdims=True)
                     == 0.0).astype(jnp.float32)

        coef = (1.0 - _MOMENTUM) * jnp.exp(ca_col * _LN_M)
        sum_term = jnp.dot(E * coef, refx_ref[...],
                           preferred_element_type=jnp.float32)
        dec = jnp.exp(cnt * _LN_M)
        delta = (dec - 1.0) * old_ref[...] + sum_term

        tw_t = jnp.exp(tl[...] - lse)
        corr = jnp.dot(tw_t * first_col, delta,
                       preferred_element_type=jnp.float32)

        out_ref[...] = wold + corr + refx_ref[...]

        diag = jnp.sum(tl[...] * (jr == jc).astype(jnp.float32),
                       axis=1, keepdims=True)
        loss_ref[...] = jnp.sum(lse - diag, axis=0,
                                keepdims=True) * (1.0 / B)


def kernel(x, ref_x, ref_types, ref_proj, conv1_w, bn1_g, bn1_b,
           conv2_w, bn2_g, bn2_b, w1, b1, w2, b2):
    B, C, H, W = x.shape
    R = w2.shape[1]
    C2 = w1.shape[1]
    P = H * W
    f32 = jnp.float32
    bf16 = jnp.bfloat16

    xf_in = x.reshape(B, C, P)

    inv = 1.0 / math.sqrt(1.0 + 1e-5)
    wc1 = jnp.transpose(conv1_w, (3, 0, 2, 1)).reshape(3 * C, 3 * C)
    wc1 = wc1.astype(bf16)
    wc2 = jnp.transpose(conv2_w, (3, 0, 2, 1)).reshape(3 * C, 3 * C)
    wc2 = wc2.astype(bf16)
    bn = jnp.stack([bn1_g * inv, bn1_b, bn2_g * inv, bn2_b], axis=1)

    G = 8
    xfeat3 = pl.pallas_call(
        lambda a, b_, c_, d_, o, s1: _conv_feat_kernel(
            a, b_, c_, d_, o, s1, H=H, W=W, C=C, G=G),
        grid=(B // G,),
        in_specs=[
            pl.BlockSpec((G, C, P), lambda i: (i, 0, 0)),
            pl.BlockSpec((3 * C, 3 * C), lambda i: (0, 0)),
            pl.BlockSpec((3 * C, 3 * C), lambda i: (0, 0)),
            pl.BlockSpec((C, 4), lambda i: (0, 0)),
        ],
        out_specs=pl.BlockSpec((G, C, 1), lambda i: (i, 0, 0)),
        out_shape=jax.ShapeDtypeStruct((B, C, 1), f32),
        scratch_shapes=[
            pltpu.VMEM((3 * C, G * P), bf16),
        ],
    )(xf_in, wc1, wc2, bn)
    x_feat = xfeat3.reshape(B, C)

    rf = ref_proj.reshape(R, C)
    J = 8

    def _mk_spec(j):
        return pl.BlockSpec((8, C), lambda i, rt: (rt[i * J + j] // 8, 0))

    old3 = pl.pallas_call(
        _gather_kernel,
        grid_spec=pltpu.PrefetchScalarGridSpec(
            num_scalar_prefetch=1,
            grid=(B // J,),
            in_specs=[_mk_spec(j) for j in range(J)],
            out_specs=pl.BlockSpec((1, J, C), lambda i, rt: (i, 0, 0)),
        ),
        out_shape=jax.ShapeDtypeStruct((B // J, J, C), f32),
    )(ref_types, *([rf] * J))
    old_rows = old3.reshape(B, C)

    K = 8192
    NC = pl.cdiv(R, K)

    rt_row = ref_types.reshape(1, B).astype(jnp.int32)
    rt_col = ref_types.reshape(B, 1).astype(jnp.int32)
    refx_flat = ref_x.reshape(B, C)
    b1r = b1.reshape(1, C2)
    b2r = b2.reshape(1, R)

    out1, loss2 = pl.pallas_call(
        lambda *a: _stream_kernel(*a, K=K, NC=NC, R=R, B=B, C=C),
        grid=(NC,),
        in_specs=[
            pl.BlockSpec((B, C), lambda i: (0, 0)),
            pl.BlockSpec((C, C2), lambda i: (0, 0)),
            pl.BlockSpec((1, C2), lambda i: (0, 0)),
            pl.BlockSpec((C2, K), lambda i: (0, i)),
            pl.BlockSpec((1, K), lambda i: (0, i)),
            pl.BlockSpec((K, C), lambda i: (i, 0)),
            pl.BlockSpec((1, B), lambda i: (0, 0)),
            pl.BlockSpec((B, 1), lambda i: (0, 0)),
            pl.BlockSpec((B, C), lambda i: (0, 0)),
            pl.BlockSpec((B, C), lambda i: (0, 0)),
        ],
        out_specs=[
            pl.BlockSpec((B, C), lambda i: (0, 0)),
            pl.BlockSpec((1, 1), lambda i: (0, 0)),
        ],
        out_shape=[
            jax.ShapeDtypeStruct((B, C), f32),
            jax.ShapeDtypeStruct((1, 1), f32),
        ],
        scratch_shapes=[
            pltpu.VMEM((B, 1), f32),
            pltpu.VMEM((B, 1), f32),
            pltpu.VMEM((B, C), f32),
            pltpu.VMEM((B, B), f32),
            pltpu.VMEM((B, C2), f32),
        ],
    )(x_feat, w1, b1r, w2, b2r, rf, rt_row, rt_col, refx_flat, old_rows)

    return (out1.reshape(B, C, 1, 1), loss2[0, 0])


# speedup vs baseline: 27.1024x; 1.0076x over previous
import math

import jax
import jax.numpy as jnp
from jax.experimental import pallas as pl
from jax.experimental.pallas import tpu as pltpu

_MOMENTUM = 0.99
_LN_M = math.log(_MOMENTUM)


def _conv_feat_kernel(x_ref, wc1_ref, wc2_ref, bn_ref, out_ref, xc,
                      *, H, W, C, G):
    P = H * W
    bf16 = jnp.bfloat16

    lane = jax.lax.broadcasted_iota(jnp.int32, (C, P), 1)
    w_id = lane % W
    m_left = (w_id >= 1).astype(jnp.float32)
    m_right = (w_id < W - 1).astype(jnp.float32)

    def build_v(v):
        up = jnp.concatenate(
            [jnp.zeros((C, W), bf16), v[:, :P - W]], axis=1)
        dn = jnp.concatenate(
            [v[:, W:], jnp.zeros((C, W), bf16)], axis=1)
        return jnp.concatenate([up, v, dn], axis=0)

    def combine_h(z):
        zl = jnp.concatenate(
            [jnp.zeros((C, 1), jnp.float32), z[0:C, :P - 1]], axis=1)
        zr = jnp.concatenate(
            [z[2 * C:3 * C, 1:], jnp.zeros((C, 1), jnp.float32)], axis=1)
        return z[C:2 * C] + m_left * zl + m_right * zr

    for g in range(G):
        xc[:, g * P:(g + 1) * P] = build_v(x_ref[g].astype(bf16))

    g1 = bn_ref[:, 0:1]
    bb1 = bn_ref[:, 1:2]
    g2 = bn_ref[:, 2:3]
    bb2 = bn_ref[:, 3:4]

    z1 = jnp.dot(wc1_ref[...], xc[...], preferred_element_type=jnp.float32)
    for g in range(G):
        f1g = combine_h(z1[:, g * P:(g + 1) * P])
        f1g = jax.nn.relu(f1g * g1 + bb1)
        xc[:, g * P:(g + 1) * P] = build_v(f1g.astype(bf16))

    z2 = jnp.dot(wc2_ref[...], xc[...], preferred_element_type=jnp.float32)
    for g in range(G):
        f2g = combine_h(z2[:, g * P:(g + 1) * P])
        f2g = jax.nn.relu(f2g * g2 + bb2)
        out_ref[g] = jnp.sum(f2g, axis=1, keepdims=True) * (1.0 / P)


def _gather_kernel(rt_ref, *refs):
    i = pl.program_id(0)
    srcs = refs[:-1]
    out_ref = refs[-1]
    for j, src in enumerate(srcs):
        r = rt_ref[i * len(srcs) + j] % 8
        out_ref[0, j] = src[pl.ds(r, 1), :][0]


def _stream_kernel(xf_ref, w1_ref, b1_ref, w2_ref, b2_ref, rf_ref,
                   rt_row_ref, rt_col_ref, refx_ref, old_ref,
                   out_ref, loss_ref,
                   m_s, s_s, acc, tl, h_s, *, K, NC, R, B, C):
    i = pl.program_id(0)

    @pl.when(i == 0)
    def _init():
        m_s[...] = jnp.full_like(m_s, -1e30)
        s_s[...] = jnp.zeros_like(s_s)
        acc[...] = jnp.zeros_like(acc)
        tl[...] = jnp.zeros_like(tl)
        h_s[...] = jax.nn.relu(
            jnp.dot(xf_ref[...], w1_ref[...],
                    preferred_element_type=jnp.float32) + b1_ref[...])

    h = h_s[...]
    logits = jnp.dot(h, w2_ref[...],
                     preferred_element_type=jnp.float32) + b2_ref[...]

    rem = R - i * K
    lane_row = jax.lax.broadcasted_iota(jnp.int32, (1, K), 1)
    logits = jnp.where(lane_row < rem, logits, -1e30)
    lane_col = jax.lax.broadcasted_iota(jnp.int32, (K, 1), 0)
    rfb = jnp.where(lane_col < rem, rf_ref[...], 0.0)

    mc = jnp.max(logits, axis=1, keepdims=True)
    m_new = jnp.maximum(m_s[...], mc)
    alpha = jnp.exp(m_s[...] - m_new)
    e = jnp.exp(logits - m_new)
    s_new = s_s[...] * alpha + jnp.sum(e, axis=1, keepdims=True)
    acc[...] = acc[...] * alpha + jnp.dot(e, rfb,
                                          preferred_element_type=jnp.float32)
    m_s[...] = m_new
    s_s[...] = s_new

    col = rt_row_ref[...] - i * K
    kio = jax.lax.broadcasted_iota(jnp.int32, (K, B), 0)
    maskT = (kio == col).astype(jnp.float32)
    tl[...] = tl[...] + jnp.dot(logits, maskT,
                                preferred_element_type=jnp.float32)

    @pl.when(i == NC - 1)
    def _finalize():
        lse = m_new + jnp.log(s_new)
        wold = acc[...] / s_new

        rt_r = rt_row_ref[...]
        rt_c = rt_col_ref[...]
        E = (rt_c == rt_r).astype(jnp.float32)
        jr = jax.lax.broadcasted_iota(jnp.int32, (B, B), 0)
        jc = jax.lax.broadcasted_iota(jnp.int32, (B, B), 1)
        after = (jr > jc).astype(jnp.float32)
        before = (jr < jc).astype(jnp.float32)

        ca_col = jnp.sum(E * after, axis=0, keepdims=True)
        cnt = jnp.sum(E, axis=1, keepdims=True)
        first_col = (jnp.sum(E * before, axis=0, keepdims=True)
                     == 0.0).astype(jnp.float32)

        coef = (1.0 - _MOMENTUM) * jnp.exp(ca_col * _LN_M)
        sum_term = jnp.dot(E * coef, refx_ref[...],
                           preferred_element_type=jnp.float32)
        dec = jnp.exp(cnt * _LN_M)
        delta = (dec - 1.0) * old_ref[...] + sum_term

        tw_t = jnp.exp(tl[...] - lse)
        corr = jnp.dot(tw_t * first_col, delta,
                       preferred_element_type=jnp.float32)

        out_ref[...] = wold + corr + refx_ref[...]

        diag = jnp.sum(tl[...] * (jr == jc).astype(jnp.float32),
                       axis=1, keepdims=True)
        loss_ref[...] = jnp.sum(lse - diag, axis=0,
                                keepdims=True) * (1.0 / B)


def kernel(x, ref_x, ref_types, ref_proj, conv1_w, bn1_g, bn1_b,
           conv2_w, bn2_g, bn2_b, w1, b1, w2, b2):
    B, C, H, W = x.shape
    R = w2.shape[1]
    C2 = w1.shape[1]
    P = H * W
    f32 = jnp.float32
    bf16 = jnp.bfloat16

    xf_in = x.reshape(B, C, P)

    inv = 1.0 / math.sqrt(1.0 + 1e-5)
    wc1 = jnp.transpose(conv1_w, (3, 0, 2, 1)).reshape(3 * C, 3 * C)
    wc1 = wc1.astype(bf16)
    wc2 = jnp.transpose(conv2_w, (3, 0, 2, 1)).reshape(3 * C, 3 * C)
    wc2 = wc2.astype(bf16)
    bn = jnp.stack([bn1_g * inv, bn1_b, bn2_g * inv, bn2_b], axis=1)

    G = 8
    xfeat3 = pl.pallas_call(
        lambda a, b_, c_, d_, o, s1: _conv_feat_kernel(
            a, b_, c_, d_, o, s1, H=H, W=W, C=C, G=G),
        grid=(B // G,),
        in_specs=[
            pl.BlockSpec((G, C, P), lambda i: (i, 0, 0)),
            pl.BlockSpec((3 * C, 3 * C), lambda i: (0, 0)),
            pl.BlockSpec((3 * C, 3 * C), lambda i: (0, 0)),
            pl.BlockSpec((C, 4), lambda i: (0, 0)),
        ],
        out_specs=pl.BlockSpec((G, C, 1), lambda i: (i, 0, 0)),
        out_shape=jax.ShapeDtypeStruct((B, C, 1), f32),
        scratch_shapes=[
            pltpu.VMEM((3 * C, G * P), bf16),
        ],
    )(xf_in, wc1, wc2, bn)
    x_feat = xfeat3.reshape(B, C)

    rf = ref_proj.reshape(R, C)
    J = 16

    def _mk_spec(j):
        return pl.BlockSpec((8, C), lambda i, rt: (rt[i * J + j] // 8, 0))

    old3 = pl.pallas_call(
        _gather_kernel,
        grid_spec=pltpu.PrefetchScalarGridSpec(
            num_scalar_prefetch=1,
            grid=(B // J,),
            in_specs=[_mk_spec(j) for j in range(J)],
            out_specs=pl.BlockSpec((1, J, C), lambda i, rt: (i, 0, 0)),
        ),
        out_shape=jax.ShapeDtypeStruct((B // J, J, C), f32),
    )(ref_types, *([rf] * J))
    old_rows = old3.reshape(B, C)

    K = 16384
    NC = pl.cdiv(R, K)

    rt_row = ref_types.reshape(1, B).astype(jnp.int32)
    rt_col = ref_types.reshape(B, 1).astype(jnp.int32)
    refx_flat = ref_x.reshape(B, C)
    b1r = b1.reshape(1, C2)
    b2r = b2.reshape(1, R)

    out1, loss2 = pl.pallas_call(
        lambda *a: _stream_kernel(*a, K=K, NC=NC, R=R, B=B, C=C),
        grid=(NC,),
        in_specs=[
            pl.BlockSpec((B, C), lambda i: (0, 0)),
            pl.BlockSpec((C, C2), lambda i: (0, 0)),
            pl.BlockSpec((1, C2), lambda i: (0, 0)),
            pl.BlockSpec((C2, K), lambda i: (0, i)),
            pl.BlockSpec((1, K), lambda i: (0, i)),
            pl.BlockSpec((K, C), lambda i: (i, 0)),
            pl.BlockSpec((1, B), lambda i: (0, 0)),
            pl.BlockSpec((B, 1), lambda i: (0, 0)),
            pl.BlockSpec((B, C), lambda i: (0, 0)),
            pl.BlockSpec((B, C), lambda i: (0, 0)),
        ],
        out_specs=[
            pl.BlockSpec((B, C), lambda i: (0, 0)),
            pl.BlockSpec((1, 1), lambda i: (0, 0)),
        ],
        out_shape=[
            jax.ShapeDtypeStruct((B, C), f32),
            jax.ShapeDtypeStruct((1, 1), f32),
        ],
        scratch_shapes=[
            pltpu.VMEM((B, 1), f32),
            pltpu.VMEM((B, 1), f32),
            pltpu.VMEM((B, C), f32),
            pltpu.VMEM((B, B), f32),
            pltpu.VMEM((B, C2), f32),
        ],
    )(x_feat, w1, b1r, w2, b2r, rf, rt_row, rt_col, refx_flat, old_rows)

    return (out1.reshape(B, C, 1, 1), loss2[0, 0])
